# baseline (speedup 1.0000x reference)
import functools

import jax
import jax.numpy as jnp
from jax import lax
from jax.experimental import pallas as pl
from jax.experimental.pallas import tpu as pltpu
from jax.experimental.pallas import tpu_sc as plsc

UNK_PREFIX = 16
NUM_CORES = 2
NUM_SUBCORES = 16
LANES = 16
REMAP_GROUP = 32


def _detile_call(dim, vocab):
    nw = NUM_CORES * NUM_SUBCORES
    blk = 128
    sb = 2
    nfull = vocab // blk
    tail = vocab - nfull * blk
    bpw = nfull // nw
    extra = nfull - bpw * nw
    while bpw % sb or ((bpw // sb) % 4) != 2 or bpw // sb < 10:
        bpw -= 1
        extra += nw
    assert extra < nw
    spw = bpw // sb
    sb_tok = sb * blk
    sb_words = sb_tok * dim
    words = blk * dim
    mesh = plsc.VectorSubcoreMesh(
        core_axis_name="c", subcore_axis_name="s",
        num_cores=NUM_CORES, num_subcores=NUM_SUBCORES)

    @functools.partial(
        pl.kernel,
        out_type=jax.ShapeDtypeStruct((vocab * dim,), jnp.float32),
        mesh=mesh,
        scratch_types=[
            pltpu.VMEM((dim, sb_tok), jnp.float32),
            pltpu.VMEM((dim, sb_tok), jnp.float32),
            pltpu.VMEM((dim, sb_tok), jnp.float32),
            pltpu.VMEM((dim, sb_tok), jnp.float32),
            pltpu.VMEM((dim, tail or LANES), jnp.float32),
            pltpu.VMEM((LANES, LANES), jnp.int32),
            pltpu.VMEM((LANES, LANES), jnp.int32),
            pltpu.VMEM((sb_words,), jnp.float32),
            pltpu.VMEM((sb_words,), jnp.float32),
            pltpu.SemaphoreType.DMA,
            pltpu.SemaphoreType.DMA,
            pltpu.SemaphoreType.DMA,
            pltpu.SemaphoreType.DMA,
            pltpu.SemaphoreType.DMA,
            pltpu.SemaphoreType.DMA,
        ],
        compiler_params=pltpu.CompilerParams(
            needs_layout_passes=False, use_tc_tiling_on_sc=True),
    )
    def detile_kernel(tab_hbm, out_hbm, b0, b1, b2, b3, blk_t, pjt, wjt,
                      r0, r1, g0, g1, g2, g3, w0, w1):
        wid = lax.axis_index("s") * NUM_CORES + lax.axis_index("c")
        sbase = wid * spw
        blks = (b0, b1, b2, b3)
        rows = (r0, r1)
        gsem = (g0, g1, g2, g3)
        wsem = (w0, w1)
        iota = lax.broadcasted_iota(jnp.int32, (LANES,), 0)
        iota_dim = dim * iota
        for j in range(LANES):
            pj = jnp.bitwise_and(iota + j, LANES - 1)
            pjt[j, :] = pj
            wjt[j, :] = iota_dim + pj

        def load(s, k):
            return pltpu.async_copy(
                tab_hbm.at[:, pl.ds((sbase + s) * sb_tok, sb_tok)],
                blks[k], gsem[k])

        def store(s, r):
            return pltpu.async_copy(
                rows[r],
                out_hbm.at[pl.ds((sbase + s) * sb_words, sb_words)], wsem[r])

        def wait_store(s, r):
            pltpu.make_async_copy(
                rows[r],
                out_hbm.at[pl.ds((sbase + s) * sb_words, sb_words)],
                wsem[r]).wait()

        def wait_load(s, k):
            pltpu.make_async_copy(
                tab_hbm.at[:, pl.ds((sbase + s) * sb_tok, sb_tok)],
                blks[k], gsem[k]).wait()

        def transpose(src, r, ntok):
            def gbody(g, carry):
                t = g * LANES
                tv = iota + t
                ts = t * dim

                def jbody(j, jc):
                    pj = pjt[j, :]
                    wj = wjt[j, :]
                    for dbase in range(0, dim, LANES):
                        v = plsc.load_gather(src, [pj + dbase, tv])
                        plsc.store_scatter(
                            rows[r], [wj + (ts + dbase)], v)
                    return jc
                lax.fori_loop(0, LANES, jbody, 0)
                return carry
            lax.fori_loop(0, ntok // LANES, gbody, 0)

        ld = {}
        st = {}
        for k in range(4):
            ld[k] = load(k, k)
        for s in range(4):
            k, r = s % 4, s % 2
            if s >= 2:
                st[s - 2].wait()
            ld[k].wait()
            transpose(blks[k], r, sb_tok)
            st[s] = store(s, r)
            ld[k] = load(s + 4, k)

        def body(i, carry):
            for k in range(4):
                s = 4 * i + k
                r = k % 2
                wait_store(s - 2, r)
                wait_load(s, k)
                transpose(blks[k], r, sb_tok)
                store(s, r)
                load(jnp.minimum(s + 4, spw - 1), k)
            return carry

        lax.fori_loop(1, (spw - 2) // 4, body, 0)

        for s in (spw - 2, spw - 1):
            k, r = s % 4, s % 2
            wait_store(s - 2, r)
            wait_load(s, k)
            transpose(blks[k], r, sb_tok)
            store(s, r)
        for s in (spw - 2, spw - 1):
            wait_store(s, s % 2)
        for k in (2, 3):
            wait_load(spw - 1, k)

        if extra:
            @pl.when(wid < extra)
            def _extras():
                c = bpw * nw + wid
                pltpu.sync_copy(tab_hbm.at[:, pl.ds(c * blk, blk)],
                                b0.at[:, pl.ds(0, blk)])
                transpose(b0, 0, blk)
                pltpu.sync_copy(r0.at[pl.ds(0, words)],
                                out_hbm.at[pl.ds(c * words, words)])

        if tail:
            @pl.when(wid == extra)
            def _tail():
                c = nfull
                pltpu.sync_copy(tab_hbm.at[:, pl.ds(c * blk, tail)], blk_t)
                transpose(blk_t, 0, tail)
                pltpu.sync_copy(r0.at[pl.ds(0, tail * dim)],
                                out_hbm.at[pl.ds(c * words, tail * dim)])

    return detile_kernel


def _emb_call(n_ids, dim, chunk):
    nw = NUM_CORES * NUM_SUBCORES
    rows_per_w = n_ids // nw
    n_chunks = rows_per_w // chunk
    assert n_chunks * chunk == rows_per_w
    remap_steps = rows_per_w // (LANES * REMAP_GROUP)
    assert remap_steps * LANES * REMAP_GROUP == rows_per_w
    mesh = plsc.VectorSubcoreMesh(
        core_axis_name="c", subcore_axis_name="s",
        num_cores=NUM_CORES, num_subcores=NUM_SUBCORES)

    @functools.partial(
        pl.kernel,
        out_type=jax.ShapeDtypeStruct((n_ids, dim), jnp.float32),
        mesh=mesh,
        scratch_types=[
            pltpu.VMEM((UNK_PREFIX,), jnp.int32),
            pltpu.VMEM((rows_per_w,), jnp.int32),
            pltpu.VMEM((chunk, dim), jnp.float32),
            pltpu.VMEM((chunk, dim), jnp.float32),
            pltpu.SemaphoreType.DMA,
            pltpu.SemaphoreType.DMA,
            pltpu.SemaphoreType.DMA,
            pltpu.SemaphoreType.DMA,
        ],
        compiler_params=pltpu.CompilerParams(
            needs_layout_passes=False, use_tc_tiling_on_sc=False),
    )
    def emb_kernel(x_hbm, table_hbm, unk_hbm, out_hbm,
                   unk_v, idx_v, rows0, rows1, g0, g1, w0, w1):
        wid = lax.axis_index("s") * NUM_CORES + lax.axis_index("c")
        base = wid * rows_per_w
        pltpu.sync_copy(unk_hbm.at[pl.ds(0, UNK_PREFIX)], unk_v)
        pltpu.sync_copy(x_hbm.at[pl.ds(base, rows_per_w)], idx_v)

        def remap_body(g, carry):
            s = g * (LANES * REMAP_GROUP)
            for i in range(REMAP_GROUP):
                v = idx_v[pl.ds(s + i * LANES, LANES)]
                inb = v < UNK_PREFIX
                m = plsc.load_gather(unk_v, [jnp.where(inb, v, 0)])
                idx_v[pl.ds(s + i * LANES, LANES)] = jnp.where(inb, m, v)
            return carry

        lax.fori_loop(0, remap_steps, remap_body, 0)

        rows = (rows0, rows1)
        gsem = (g0, g1)
        wsem = (w0, w1)

        def gather(c, k):
            return pltpu.async_copy(
                table_hbm.at[idx_v.at[pl.ds(c * chunk, chunk)]],
                rows[k], gsem[k])

        def writeout(c, k):
            return pltpu.async_copy(
                rows[k], out_hbm.at[pl.ds(base + c * chunk, chunk)], wsem[k])

        gd = {0: gather(0, 0)}
        wd = {}
        for c in range(n_chunks):
            k = c % 2
            if c + 1 < n_chunks:
                if c >= 1:
                    wd[c - 1].wait()
                gd[c + 1] = gather(c + 1, 1 - k)
            gd[c].wait()
            wd[c] = writeout(c, k)
        wd[n_chunks - 2].wait()
        wd[n_chunks - 1].wait()

    return emb_kernel


def kernel(x, table, unkmap):
    b, l = x.shape
    vocab, dim = table.shape
    n_ids = b * l
    xf = x.reshape(n_ids)
    table_t = jnp.swapaxes(table, 0, 1)
    flat = _detile_call(dim, vocab)(table_t)
    table_rm = flat.reshape(vocab, dim)
    out = _emb_call(n_ids, dim, chunk=512)(xf, table_rm, unkmap)
    return out.reshape(b, l, dim)


# speedup vs baseline: 1.2413x; 1.2413x over previous
import functools

import jax
import jax.numpy as jnp
from jax import lax
from jax.experimental import pallas as pl
from jax.experimental.pallas import tpu as pltpu
from jax.experimental.pallas import tpu_sc as plsc

UNK_PREFIX = 16
NUM_CORES = 2
NUM_SUBCORES = 16
LANES = 16
REMAP_GROUP = 32


def _detile_call(dim, vocab):
    nw = NUM_CORES * NUM_SUBCORES
    blk = 128
    sb = 2
    nfull = vocab // blk
    tail = vocab - nfull * blk
    bpw = nfull // nw
    extra = nfull - bpw * nw
    while bpw % sb or ((bpw // sb) % 4) != 2 or bpw // sb < 10:
        bpw -= 1
        extra += nw
    assert extra < nw
    spw = bpw // sb
    sb_tok = sb * blk
    sb_words = sb_tok * dim
    words = blk * dim
    mesh = plsc.VectorSubcoreMesh(
        core_axis_name="c", subcore_axis_name="s",
        num_cores=NUM_CORES, num_subcores=NUM_SUBCORES)

    @functools.partial(
        pl.kernel,
        out_type=jax.ShapeDtypeStruct((vocab * dim,), jnp.float32),
        mesh=mesh,
        scratch_types=[
            pltpu.VMEM((dim, sb_tok), jnp.float32),
            pltpu.VMEM((dim, sb_tok), jnp.float32),
            pltpu.VMEM((dim, sb_tok), jnp.float32),
            pltpu.VMEM((dim, sb_tok), jnp.float32),
            pltpu.VMEM((dim, tail or LANES), jnp.float32),
            pltpu.VMEM((LANES, LANES), jnp.int32),
            pltpu.VMEM((LANES, LANES), jnp.int32),
            pltpu.VMEM((sb_words,), jnp.float32),
            pltpu.VMEM((sb_words,), jnp.float32),
            pltpu.SemaphoreType.DMA,
            pltpu.SemaphoreType.DMA,
            pltpu.SemaphoreType.DMA,
            pltpu.SemaphoreType.DMA,
            pltpu.SemaphoreType.DMA,
            pltpu.SemaphoreType.DMA,
        ],
        compiler_params=pltpu.CompilerParams(
            needs_layout_passes=False, use_tc_tiling_on_sc=True),
    )
    def detile_kernel(tab_hbm, out_hbm, b0, b1, b2, b3, blk_t, pjt, wjt,
                      r0, r1, g0, g1, g2, g3, w0, w1):
        wid = lax.axis_index("s") * NUM_CORES + lax.axis_index("c")
        sbase = wid * spw
        blks = (b0, b1, b2, b3)
        rows = (r0, r1)
        gsem = (g0, g1, g2, g3)
        wsem = (w0, w1)
        iota = lax.broadcasted_iota(jnp.int32, (LANES,), 0)
        iota_dim = dim * iota
        for j in range(LANES):
            pj = jnp.bitwise_and(iota + j, LANES - 1)
            pjt[j, :] = pj
            wjt[j, :] = iota_dim + pj

        def load(s, k):
            return pltpu.async_copy(
                tab_hbm.at[:, pl.ds((sbase + s) * sb_tok, sb_tok)],
                blks[k], gsem[k])

        def store(s, r):
            return pltpu.async_copy(
                rows[r],
                out_hbm.at[pl.ds((sbase + s) * sb_words, sb_words)], wsem[r])

        def wait_store(s, r):
            pltpu.make_async_copy(
                rows[r],
                out_hbm.at[pl.ds((sbase + s) * sb_words, sb_words)],
                wsem[r]).wait()

        def wait_load(s, k):
            pltpu.make_async_copy(
                tab_hbm.at[:, pl.ds((sbase + s) * sb_tok, sb_tok)],
                blks[k], gsem[k]).wait()

        def transpose(src, r, ntok):
            def jbody(j, jc):
                pj = pjt[j, :]
                wj = wjt[j, :]
                for g in range(ntok // LANES):
                    tv = iota + g * LANES
                    for dbase in range(0, dim, LANES):
                        v = plsc.load_gather(src, [pj + dbase, tv])
                        plsc.store_scatter(
                            rows[r], [wj + (g * LANES * dim + dbase)], v)
                return jc
            lax.fori_loop(0, LANES, jbody, 0)

        ld = {}
        st = {}
        for k in range(4):
            ld[k] = load(k, k)
        for s in range(4):
            k, r = s % 4, s % 2
            if s >= 2:
                st[s - 2].wait()
            ld[k].wait()
            transpose(blks[k], r, sb_tok)
            st[s] = store(s, r)
            ld[k] = load(s + 4, k)

        def body(i, carry):
            for k in range(4):
                s = 4 * i + k
                r = k % 2
                wait_store(s - 2, r)
                wait_load(s, k)
                transpose(blks[k], r, sb_tok)
                store(s, r)
                load(jnp.minimum(s + 4, spw - 1), k)
            return carry

        lax.fori_loop(1, (spw - 2) // 4, body, 0)

        for s in (spw - 2, spw - 1):
            k, r = s % 4, s % 2
            wait_store(s - 2, r)
            wait_load(s, k)
            transpose(blks[k], r, sb_tok)
            store(s, r)
        for s in (spw - 2, spw - 1):
            wait_store(s, s % 2)
        for k in (2, 3):
            wait_load(spw - 1, k)

        if extra:
            @pl.when(wid < extra)
            def _extras():
                c = bpw * nw + wid
                pltpu.sync_copy(tab_hbm.at[:, pl.ds(c * blk, blk)],
                                b0.at[:, pl.ds(0, blk)])
                transpose(b0, 0, blk)
                pltpu.sync_copy(r0.at[pl.ds(0, words)],
                                out_hbm.at[pl.ds(c * words, words)])

        if tail:
            @pl.when(wid == extra)
            def _tail():
                c = nfull
                pltpu.sync_copy(tab_hbm.at[:, pl.ds(c * blk, tail)], blk_t)
                transpose(blk_t, 0, tail)
                pltpu.sync_copy(r0.at[pl.ds(0, tail * dim)],
                                out_hbm.at[pl.ds(c * words, tail * dim)])

    return detile_kernel


def _emb_call(n_ids, dim, chunk):
    nw = NUM_CORES * NUM_SUBCORES
    rows_per_w = n_ids // nw
    n_chunks = rows_per_w // chunk
    assert n_chunks * chunk == rows_per_w
    remap_steps = rows_per_w // (LANES * REMAP_GROUP)
    assert remap_steps * LANES * REMAP_GROUP == rows_per_w
    mesh = plsc.VectorSubcoreMesh(
        core_axis_name="c", subcore_axis_name="s",
        num_cores=NUM_CORES, num_subcores=NUM_SUBCORES)

    @functools.partial(
        pl.kernel,
        out_type=jax.ShapeDtypeStruct((n_ids, dim), jnp.float32),
        mesh=mesh,
        scratch_types=[
            pltpu.VMEM((UNK_PREFIX,), jnp.int32),
            pltpu.VMEM((rows_per_w,), jnp.int32),
            pltpu.VMEM((chunk, dim), jnp.float32),
            pltpu.VMEM((chunk, dim), jnp.float32),
            pltpu.SemaphoreType.DMA,
            pltpu.SemaphoreType.DMA,
            pltpu.SemaphoreType.DMA,
            pltpu.SemaphoreType.DMA,
        ],
        compiler_params=pltpu.CompilerParams(
            needs_layout_passes=False, use_tc_tiling_on_sc=False),
    )
    def emb_kernel(x_hbm, table_hbm, unk_hbm, out_hbm,
                   unk_v, idx_v, rows0, rows1, g0, g1, w0, w1):
        wid = lax.axis_index("s") * NUM_CORES + lax.axis_index("c")
        base = wid * rows_per_w
        pltpu.sync_copy(unk_hbm.at[pl.ds(0, UNK_PREFIX)], unk_v)
        pltpu.sync_copy(x_hbm.at[pl.ds(base, rows_per_w)], idx_v)

        def remap_body(g, carry):
            s = g * (LANES * REMAP_GROUP)
            for i in range(REMAP_GROUP):
                v = idx_v[pl.ds(s + i * LANES, LANES)]
                inb = v < UNK_PREFIX
                m = plsc.load_gather(unk_v, [jnp.where(inb, v, 0)])
                idx_v[pl.ds(s + i * LANES, LANES)] = jnp.where(inb, m, v)
            return carry

        lax.fori_loop(0, remap_steps, remap_body, 0)

        rows = (rows0, rows1)
        gsem = (g0, g1)
        wsem = (w0, w1)

        def gather(c, k):
            return pltpu.async_copy(
                table_hbm.at[idx_v.at[pl.ds(c * chunk, chunk)]],
                rows[k], gsem[k])

        def writeout(c, k):
            return pltpu.async_copy(
                rows[k], out_hbm.at[pl.ds(base + c * chunk, chunk)], wsem[k])

        gd = {0: gather(0, 0)}
        wd = {}
        for c in range(n_chunks):
            k = c % 2
            if c + 1 < n_chunks:
                if c >= 1:
                    wd[c - 1].wait()
                gd[c + 1] = gather(c + 1, 1 - k)
            gd[c].wait()
            wd[c] = writeout(c, k)
        wd[n_chunks - 2].wait()
        wd[n_chunks - 1].wait()

    return emb_kernel


def kernel(x, table, unkmap):
    b, l = x.shape
    vocab, dim = table.shape
    n_ids = b * l
    xf = x.reshape(n_ids)
    table_t = jnp.swapaxes(table, 0, 1)
    flat = _detile_call(dim, vocab)(table_t)
    table_rm = flat.reshape(vocab, dim)
    out = _emb_call(n_ids, dim, chunk=512)(xf, table_rm, unkmap)
    return out.reshape(b, l, dim)


# speedup vs baseline: 1.2482x; 1.0056x over previous
import functools

import jax
import jax.numpy as jnp
from jax import lax
from jax.experimental import pallas as pl
from jax.experimental.pallas import tpu as pltpu
from jax.experimental.pallas import tpu_sc as plsc

UNK_PREFIX = 16
NUM_CORES = 2
NUM_SUBCORES = 16
LANES = 16
REMAP_GROUP = 32


def _detile_call(dim, vocab):
    nw = NUM_CORES * NUM_SUBCORES
    blk = 128
    sb = 2
    nfull = vocab // blk
    tail = vocab - nfull * blk
    bpw = nfull // nw
    extra = nfull - bpw * nw
    while bpw % sb or ((bpw // sb) % 4) != 2 or bpw // sb < 10:
        bpw -= 1
        extra += nw
    assert extra < nw
    spw = bpw // sb
    sb_tok = sb * blk
    sb_words = sb_tok * dim
    words = blk * dim
    mesh = plsc.VectorSubcoreMesh(
        core_axis_name="c", subcore_axis_name="s",
        num_cores=NUM_CORES, num_subcores=NUM_SUBCORES)

    @functools.partial(
        pl.kernel,
        out_type=jax.ShapeDtypeStruct((vocab * dim,), jnp.float32),
        mesh=mesh,
        scratch_types=[
            pltpu.VMEM((dim, sb_tok), jnp.float32),
            pltpu.VMEM((dim, sb_tok), jnp.float32),
            pltpu.VMEM((dim, sb_tok), jnp.float32),
            pltpu.VMEM((dim, sb_tok), jnp.float32),
            pltpu.VMEM((dim, tail or LANES), jnp.float32),
            pltpu.VMEM((LANES, LANES), jnp.int32),
            pltpu.VMEM((LANES, LANES), jnp.int32),
            pltpu.VMEM((sb_words,), jnp.float32),
            pltpu.VMEM((sb_words,), jnp.float32),
            pltpu.SemaphoreType.DMA,
            pltpu.SemaphoreType.DMA,
            pltpu.SemaphoreType.DMA,
            pltpu.SemaphoreType.DMA,
            pltpu.SemaphoreType.DMA,
            pltpu.SemaphoreType.DMA,
        ],
        compiler_params=pltpu.CompilerParams(
            needs_layout_passes=False, use_tc_tiling_on_sc=True),
    )
    def detile_kernel(tab_hbm, out_hbm, b0, b1, b2, b3, blk_t, pjt, wjt,
                      r0, r1, g0, g1, g2, g3, w0, w1):
        wid = lax.axis_index("s") * NUM_CORES + lax.axis_index("c")
        sbase = wid * spw
        blks = (b0, b1, b2, b3)
        rows = (r0, r1)
        gsem = (g0, g1, g2, g3)
        wsem = (w0, w1)
        iota = lax.broadcasted_iota(jnp.int32, (LANES,), 0)
        iota_dim = dim * iota
        for j in range(LANES):
            pj = jnp.bitwise_and(iota + j, LANES - 1)
            pjt[j, :] = pj
            wjt[j, :] = iota_dim + pj

        def load(s, k):
            return pltpu.async_copy(
                tab_hbm.at[:, pl.ds((sbase + s) * sb_tok, sb_tok)],
                blks[k], gsem[k])

        def store(s, r):
            return pltpu.async_copy(
                rows[r],
                out_hbm.at[pl.ds((sbase + s) * sb_words, sb_words)], wsem[r])

        def wait_store(s, r):
            pltpu.make_async_copy(
                rows[r],
                out_hbm.at[pl.ds((sbase + s) * sb_words, sb_words)],
                wsem[r]).wait()

        def wait_load(s, k):
            pltpu.make_async_copy(
                tab_hbm.at[:, pl.ds((sbase + s) * sb_tok, sb_tok)],
                blks[k], gsem[k]).wait()

        def transpose(src, r, ntok):
            def jbody(j, jc):
                pj = pjt[j, :]
                wj = wjt[j, :]
                for g in range(ntok // LANES):
                    tv = iota + g * LANES
                    for dbase in range(0, dim, LANES):
                        v = plsc.load_gather(src, [pj + dbase, tv])
                        plsc.store_scatter(
                            rows[r], [wj + (g * LANES * dim + dbase)], v)
                return jc
            lax.fori_loop(0, LANES, jbody, 0)

        ld = {}
        st = {}
        for k in range(4):
            ld[k] = load(k, k)
        for s in range(4):
            k, r = s % 4, s % 2
            if s >= 2:
                st[s - 2].wait()
            ld[k].wait()
            transpose(blks[k], r, sb_tok)
            st[s] = store(s, r)
            ld[k] = load(s + 4, k)

        def body(i, carry):
            for k in range(4):
                s = 4 * i + k
                r = k % 2
                wait_store(s - 2, r)
                wait_load(s, k)
                transpose(blks[k], r, sb_tok)
                store(s, r)
                load(jnp.minimum(s + 4, spw - 1), k)
            return carry

        lax.fori_loop(1, (spw - 2) // 4, body, 0)

        for s in (spw - 2, spw - 1):
            k, r = s % 4, s % 2
            wait_store(s - 2, r)
            wait_load(s, k)
            transpose(blks[k], r, sb_tok)
            store(s, r)
        for s in (spw - 2, spw - 1):
            wait_store(s, s % 2)
        for k in (2, 3):
            wait_load(spw - 1, k)

        if extra:
            @pl.when(wid < extra)
            def _extras():
                c = bpw * nw + wid
                pltpu.sync_copy(tab_hbm.at[:, pl.ds(c * blk, blk)],
                                b0.at[:, pl.ds(0, blk)])
                transpose(b0, 0, blk)
                pltpu.sync_copy(r0.at[pl.ds(0, words)],
                                out_hbm.at[pl.ds(c * words, words)])

        if tail:
            @pl.when(wid == extra)
            def _tail():
                c = nfull
                pltpu.sync_copy(tab_hbm.at[:, pl.ds(c * blk, tail)], blk_t)
                transpose(blk_t, 0, tail)
                pltpu.sync_copy(r0.at[pl.ds(0, tail * dim)],
                                out_hbm.at[pl.ds(c * words, tail * dim)])

    return detile_kernel


def _emb_call(bsz, seq, dim, chunk):
    nw = NUM_CORES * NUM_SUBCORES
    cpl = bsz // chunk
    cpw_l = cpl // nw
    assert cpw_l * nw == cpl and cpw_l == 2
    n_chunks = seq * cpw_l
    mesh = plsc.VectorSubcoreMesh(
        core_axis_name="c", subcore_axis_name="s",
        num_cores=NUM_CORES, num_subcores=NUM_SUBCORES)

    @functools.partial(
        pl.kernel,
        out_type=jax.ShapeDtypeStruct((seq * dim, bsz), jnp.float32),
        mesh=mesh,
        scratch_types=[
            pltpu.VMEM((UNK_PREFIX,), jnp.int32),
            pltpu.VMEM((chunk,), jnp.int32),
            pltpu.VMEM((chunk,), jnp.int32),
            pltpu.VMEM((chunk, dim), jnp.float32),
            pltpu.VMEM((chunk, dim), jnp.float32),
            pltpu.VMEM((dim, chunk), jnp.float32),
            pltpu.VMEM((dim, chunk), jnp.float32),
            pltpu.VMEM((LANES, LANES), jnp.int32),
            pltpu.SemaphoreType.DMA,
            pltpu.SemaphoreType.DMA,
            pltpu.SemaphoreType.DMA,
            pltpu.SemaphoreType.DMA,
            pltpu.SemaphoreType.DMA,
            pltpu.SemaphoreType.DMA,
        ],
        compiler_params=pltpu.CompilerParams(
            needs_layout_passes=False, use_tc_tiling_on_sc=False),
    )
    def emb_kernel(xt_hbm, table_hbm, unk_hbm, out_hbm,
                   unk_v, i0, i1, r0, r1, t0, t1, pjt,
                   gi0, gi1, gg0, gg1, gw0, gw1):
        wid = lax.axis_index("s") * NUM_CORES + lax.axis_index("c")
        iota = lax.broadcasted_iota(jnp.int32, (LANES,), 0)
        for j in range(LANES):
            pjt[j, :] = jnp.bitwise_and(iota + j, LANES - 1)
        pltpu.sync_copy(unk_hbm.at[pl.ds(0, UNK_PREFIX)], unk_v)
        idx = (i0, i1)
        rows = (r0, r1)
        rowt = (t0, t1)
        isem = (gi0, gi1)
        gsem = (gg0, gg1)
        wsem = (gw0, gw1)

        def src_off(c_l, k):
            return c_l * bsz + (wid * cpw_l + k) * chunk

        def idx_load(c_l, k):
            return pltpu.async_copy(
                xt_hbm.at[pl.ds(src_off(c_l, k), chunk)], idx[k], isem[k])

        def wait_idx(c_l, k):
            pltpu.make_async_copy(
                xt_hbm.at[pl.ds(src_off(c_l, k), chunk)], idx[k],
                isem[k]).wait()

        def remap(k):
            for i in range(chunk // LANES):
                v = idx[k][pl.ds(i * LANES, LANES)]
                inb = v < UNK_PREFIX
                m = plsc.load_gather(unk_v, [jnp.where(inb, v, 0)])
                idx[k][pl.ds(i * LANES, LANES)] = jnp.where(inb, m, v)

        def gather(k):
            return pltpu.async_copy(table_hbm.at[idx[k]], rows[k], gsem[k])

        def wait_gather(k):
            pltpu.make_async_copy(
                table_hbm.at[idx[k]], rows[k], gsem[k]).wait()

        def writeout(c_l, k):
            return pltpu.async_copy(
                rowt[k],
                out_hbm.at[pl.ds(c_l * dim, dim),
                           pl.ds((wid * cpw_l + k) * chunk, chunk)], wsem[k])

        def wait_write(c_l, k):
            pltpu.make_async_copy(
                rowt[k],
                out_hbm.at[pl.ds(c_l * dim, dim),
                           pl.ds((wid * cpw_l + k) * chunk, chunk)],
                wsem[k]).wait()

        def transpose(k):
            def jbody(j, jc):
                pj = pjt[j, :]
                for g in range(chunk // LANES):
                    tv = iota + g * LANES
                    for dbase in range(0, dim, LANES):
                        dv = pj + dbase
                        v = plsc.load_gather(rows[k], [tv, dv])
                        plsc.store_scatter(rowt[k], [dv, tv], v)
                return jc
            lax.fori_loop(0, LANES, jbody, 0)

        idx_load(0, 0).wait()
        remap(0)
        gather(0)
        idx_load(0, 1).wait()
        remap(1)

        wait_gather(0)
        gather(1)
        transpose(0)
        writeout(0, 0)
        idx_load(1, 0)
        wait_gather(1)
        wait_idx(1, 0)
        remap(0)
        gather(0)
        transpose(1)
        writeout(0, 1)
        idx_load(1, 1)

        def body(i, carry):
            for k in (0, 1):
                wait_gather(k)
                nk = 1 - k
                nl = i + (1 if k == 1 else 0)
                wait_idx(nl, nk)
                remap(nk)
                gather(nk)
                wait_write(i - 1, k)
                transpose(k)
                writeout(i, k)
                idx_load(i + 1, k)
            return carry

        lax.fori_loop(1, seq - 1, body, 0)

        wait_gather(0)
        wait_idx(seq - 1, 1)
        remap(1)
        gather(1)
        wait_write(seq - 2, 0)
        transpose(0)
        writeout(seq - 1, 0)
        wait_gather(1)
        wait_write(seq - 2, 1)
        transpose(1)
        writeout(seq - 1, 1)
        wait_write(seq - 1, 0)
        wait_write(seq - 1, 1)

    return emb_kernel


def kernel(x, table, unkmap):
    b, l = x.shape
    vocab, dim = table.shape
    n_ids = b * l
    xt = jnp.swapaxes(x, 0, 1).reshape(n_ids)
    table_t = jnp.swapaxes(table, 0, 1)
    flat = _detile_call(dim, vocab)(table_t)
    table_rm = flat.reshape(vocab, dim)
    out3 = _emb_call(b, l, dim, chunk=256)(xt, table_rm, unkmap)
    return jnp.transpose(out3.reshape(l, dim, b), (2, 0, 1))


# speedup vs baseline: 2.0259x; 1.6231x over previous
import functools

import jax
import jax.numpy as jnp
from jax import lax
from jax.experimental import pallas as pl
from jax.experimental.pallas import tpu as pltpu
from jax.experimental.pallas import tpu_sc as plsc

UNK_PREFIX = 16
NUM_CORES = 2
NUM_SUBCORES = 16
LANES = 16
REMAP_GROUP = 32


def _detile_call(dim, vocab):
    nw = NUM_CORES * NUM_SUBCORES
    blk = 128
    sb = 2
    nfull = vocab // blk
    tail = vocab - nfull * blk
    bpw = nfull // nw
    extra = nfull - bpw * nw
    while bpw % sb or ((bpw // sb) % 4) != 2 or bpw // sb < 10:
        bpw -= 1
        extra += nw
    assert extra < nw
    spw = bpw // sb
    sb_tok = sb * blk
    sb_words = sb_tok * dim
    words = blk * dim
    mesh = plsc.VectorSubcoreMesh(
        core_axis_name="c", subcore_axis_name="s",
        num_cores=NUM_CORES, num_subcores=NUM_SUBCORES)

    @functools.partial(
        pl.kernel,
        out_type=jax.ShapeDtypeStruct((vocab * dim,), jnp.float32),
        mesh=mesh,
        scratch_types=[
            pltpu.VMEM((dim, sb_tok), jnp.float32),
            pltpu.VMEM((dim, sb_tok), jnp.float32),
            pltpu.VMEM((dim, sb_tok), jnp.float32),
            pltpu.VMEM((dim, sb_tok), jnp.float32),
            pltpu.VMEM((dim, tail or LANES), jnp.float32),
            pltpu.VMEM((LANES, LANES), jnp.int32),
            pltpu.VMEM((LANES, LANES), jnp.int32),
            pltpu.VMEM((sb_words,), jnp.float32),
            pltpu.VMEM((sb_words,), jnp.float32),
            pltpu.SemaphoreType.DMA,
            pltpu.SemaphoreType.DMA,
            pltpu.SemaphoreType.DMA,
            pltpu.SemaphoreType.DMA,
            pltpu.SemaphoreType.DMA,
            pltpu.SemaphoreType.DMA,
        ],
        compiler_params=pltpu.CompilerParams(
            needs_layout_passes=False, use_tc_tiling_on_sc=True),
    )
    def detile_kernel(tab_hbm, out_hbm, b0, b1, b2, b3, blk_t, pjt, wjt,
                      r0, r1, g0, g1, g2, g3, w0, w1):
        wid = lax.axis_index("s") * NUM_CORES + lax.axis_index("c")
        sbase = wid * spw
        blks = (b0, b1, b2, b3)
        rows = (r0, r1)
        gsem = (g0, g1, g2, g3)
        wsem = (w0, w1)
        iota = lax.broadcasted_iota(jnp.int32, (LANES,), 0)
        iota_dim = dim * iota
        for j in range(LANES):
            pj = jnp.bitwise_and(iota + j, LANES - 1)
            pjt[j, :] = pj
            wjt[j, :] = iota_dim + pj

        def load(s, k):
            return pltpu.async_copy(
                tab_hbm.at[:, pl.ds((sbase + s) * sb_tok, sb_tok)],
                blks[k], gsem[k])

        def store(s, r):
            return pltpu.async_copy(
                rows[r],
                out_hbm.at[pl.ds((sbase + s) * sb_words, sb_words)], wsem[r])

        def wait_store(s, r):
            pltpu.make_async_copy(
                rows[r],
                out_hbm.at[pl.ds((sbase + s) * sb_words, sb_words)],
                wsem[r]).wait()

        def wait_load(s, k):
            pltpu.make_async_copy(
                tab_hbm.at[:, pl.ds((sbase + s) * sb_tok, sb_tok)],
                blks[k], gsem[k]).wait()

        def transpose(src, r, ntok):
            def jbody(j, jc):
                pj = pjt[j, :]
                wj = wjt[j, :]
                for g in range(ntok // LANES):
                    tv = iota + g * LANES
                    vs = [plsc.load_gather(src, [pj + dbase, tv])
                          for dbase in range(0, dim, LANES)]
                    for di, dbase in enumerate(range(0, dim, LANES)):
                        plsc.store_scatter(
                            rows[r], [wj + (g * LANES * dim + dbase)], vs[di])
                return jc
            lax.fori_loop(0, LANES, jbody, 0)

        ld = {}
        st = {}
        for k in range(4):
            ld[k] = load(k, k)
        for s in range(4):
            k, r = s % 4, s % 2
            if s >= 2:
                st[s - 2].wait()
            ld[k].wait()
            transpose(blks[k], r, sb_tok)
            st[s] = store(s, r)
            ld[k] = load(s + 4, k)

        def body(i, carry):
            for k in range(4):
                s = 4 * i + k
                r = k % 2
                wait_store(s - 2, r)
                wait_load(s, k)
                transpose(blks[k], r, sb_tok)
                store(s, r)
                load(jnp.minimum(s + 4, spw - 1), k)
            return carry

        lax.fori_loop(1, (spw - 2) // 4, body, 0)

        for s in (spw - 2, spw - 1):
            k, r = s % 4, s % 2
            wait_store(s - 2, r)
            wait_load(s, k)
            transpose(blks[k], r, sb_tok)
            store(s, r)
        for s in (spw - 2, spw - 1):
            wait_store(s, s % 2)
        for k in (2, 3):
            wait_load(spw - 1, k)

        if extra:
            @pl.when(wid < extra)
            def _extras():
                c = bpw * nw + wid
                pltpu.sync_copy(tab_hbm.at[:, pl.ds(c * blk, blk)],
                                b0.at[:, pl.ds(0, blk)])
                transpose(b0, 0, blk)
                pltpu.sync_copy(r0.at[pl.ds(0, words)],
                                out_hbm.at[pl.ds(c * words, words)])

        if tail:
            @pl.when(wid == extra)
            def _tail():
                c = nfull
                pltpu.sync_copy(tab_hbm.at[:, pl.ds(c * blk, tail)], blk_t)
                transpose(blk_t, 0, tail)
                pltpu.sync_copy(r0.at[pl.ds(0, tail * dim)],
                                out_hbm.at[pl.ds(c * words, tail * dim)])

    return detile_kernel


def _emb_call(bsz, seq, dim, chunk):
    nw = NUM_CORES * NUM_SUBCORES
    cpl = bsz // chunk
    cpw_l = cpl // nw
    assert cpw_l * nw == cpl and cpw_l == 2
    n_chunks = seq * cpw_l
    mesh = plsc.VectorSubcoreMesh(
        core_axis_name="c", subcore_axis_name="s",
        num_cores=NUM_CORES, num_subcores=NUM_SUBCORES)

    @functools.partial(
        pl.kernel,
        out_type=jax.ShapeDtypeStruct((seq * dim, bsz), jnp.float32),
        mesh=mesh,
        scratch_types=[
            pltpu.VMEM((UNK_PREFIX,), jnp.int32),
            pltpu.VMEM((chunk,), jnp.int32),
            pltpu.VMEM((chunk,), jnp.int32),
            pltpu.VMEM((chunk, dim), jnp.float32),
            pltpu.VMEM((chunk, dim), jnp.float32),
            pltpu.VMEM((dim, chunk), jnp.float32),
            pltpu.VMEM((dim, chunk), jnp.float32),
            pltpu.VMEM((LANES, LANES), jnp.int32),
            pltpu.SemaphoreType.DMA,
            pltpu.SemaphoreType.DMA,
            pltpu.SemaphoreType.DMA,
            pltpu.SemaphoreType.DMA,
            pltpu.SemaphoreType.DMA,
            pltpu.SemaphoreType.DMA,
        ],
        compiler_params=pltpu.CompilerParams(
            needs_layout_passes=False, use_tc_tiling_on_sc=False),
    )
    def emb_kernel(xt_hbm, table_hbm, unk_hbm, out_hbm,
                   unk_v, i0, i1, r0, r1, t0, t1, pjt,
                   gi0, gi1, gg0, gg1, gw0, gw1):
        wid = lax.axis_index("s") * NUM_CORES + lax.axis_index("c")
        iota = lax.broadcasted_iota(jnp.int32, (LANES,), 0)
        for j in range(LANES):
            pjt[j, :] = jnp.bitwise_and(iota + j, LANES - 1)
        pltpu.sync_copy(unk_hbm.at[pl.ds(0, UNK_PREFIX)], unk_v)
        idx = (i0, i1)
        rows = (r0, r1)
        rowt = (t0, t1)
        isem = (gi0, gi1)
        gsem = (gg0, gg1)
        wsem = (gw0, gw1)

        def src_off(c_l, k):
            return c_l * bsz + (wid * cpw_l + k) * chunk

        def idx_load(c_l, k):
            return pltpu.async_copy(
                xt_hbm.at[pl.ds(src_off(c_l, k), chunk)], idx[k], isem[k])

        def wait_idx(c_l, k):
            pltpu.make_async_copy(
                xt_hbm.at[pl.ds(src_off(c_l, k), chunk)], idx[k],
                isem[k]).wait()

        def remap(k):
            for i in range(chunk // LANES):
                v = idx[k][pl.ds(i * LANES, LANES)]
                inb = v < UNK_PREFIX
                m = plsc.load_gather(unk_v, [jnp.where(inb, v, 0)])
                idx[k][pl.ds(i * LANES, LANES)] = jnp.where(inb, m, v)

        def gather(k):
            return pltpu.async_copy(table_hbm.at[idx[k]], rows[k], gsem[k])

        def wait_gather(k):
            pltpu.make_async_copy(
                table_hbm.at[idx[k]], rows[k], gsem[k]).wait()

        def writeout(c_l, k):
            return pltpu.async_copy(
                rowt[k],
                out_hbm.at[pl.ds(c_l * dim, dim),
                           pl.ds((wid * cpw_l + k) * chunk, chunk)], wsem[k])

        def wait_write(c_l, k):
            pltpu.make_async_copy(
                rowt[k],
                out_hbm.at[pl.ds(c_l * dim, dim),
                           pl.ds((wid * cpw_l + k) * chunk, chunk)],
                wsem[k]).wait()

        def transpose(k):
            def jbody(j, jc):
                pj = pjt[j, :]
                for g in range(chunk // LANES):
                    tv = iota + g * LANES
                    dvs = [pj + dbase for dbase in range(0, dim, LANES)]
                    vs = [plsc.load_gather(rows[k], [tv, dv]) for dv in dvs]
                    for di, dv in enumerate(dvs):
                        plsc.store_scatter(rowt[k], [dv, tv], vs[di])
                return jc
            lax.fori_loop(0, LANES, jbody, 0)

        idx_load(0, 0).wait()
        remap(0)
        gather(0)
        idx_load(0, 1).wait()
        remap(1)

        wait_gather(0)
        gather(1)
        transpose(0)
        writeout(0, 0)
        idx_load(1, 0)
        wait_gather(1)
        wait_idx(1, 0)
        remap(0)
        gather(0)
        transpose(1)
        writeout(0, 1)
        idx_load(1, 1)

        def body(i, carry):
            for k in (0, 1):
                wait_gather(k)
                nk = 1 - k
                nl = i + (1 if k == 1 else 0)
                wait_idx(nl, nk)
                remap(nk)
                gather(nk)
                wait_write(i - 1, k)
                transpose(k)
                writeout(i, k)
                idx_load(i + 1, k)
            return carry

        lax.fori_loop(1, seq - 1, body, 0)

        wait_gather(0)
        wait_idx(seq - 1, 1)
        remap(1)
        gather(1)
        wait_write(seq - 2, 0)
        transpose(0)
        writeout(seq - 1, 0)
        wait_gather(1)
        wait_write(seq - 2, 1)
        transpose(1)
        writeout(seq - 1, 1)
        wait_write(seq - 1, 0)
        wait_write(seq - 1, 1)

    return emb_kernel


def kernel(x, table, unkmap):
    b, l = x.shape
    vocab, dim = table.shape
    n_ids = b * l
    xt = jnp.swapaxes(x, 0, 1).reshape(n_ids)
    table_t = jnp.swapaxes(table, 0, 1)
    flat = _detile_call(dim, vocab)(table_t)
    table_rm = flat.reshape(vocab, dim)
    out3 = _emb_call(b, l, dim, chunk=256)(xt, table_rm, unkmap)
    return jnp.transpose(out3.reshape(l, dim, b), (2, 0, 1))
